# baseline (speedup 1.0000x reference)
import functools

import jax
import jax.numpy as jnp
import numpy as np
from jax import lax
from jax.experimental import pallas as pl
from jax.experimental.pallas import tpu as pltpu
from jax.experimental.pallas import tpu_sc as plsc
from jax.scipy.special import ndtri

D = 64
NW = 32
HC = 4
FRAC = np.float32(0.999 * 63 - 62)


def _sc_body(idx_hbm, w_hbm, mvs_hbm, flut_hbm, out_hbm,
             idx_all, idx_cmp, rows_v0, rows_v1, out_v0, out_v1,
             mvs_v, flut_v, sem_g0, sem_g1, sem_o0, sem_o1, hpw, hist):
    wid = lax.axis_index("s") * 2 + lax.axis_index("c")
    pltpu.sync_copy(mvs_hbm, mvs_v)
    pltpu.sync_copy(flut_hbm, flut_v)
    iota16 = lax.iota(jnp.int32, 16)
    zero16 = iota16 * 0
    one16 = zero16 + 1
    rows_vs = (rows_v0, rows_v1)
    out_vs = (out_v0, out_v1)
    sem_gs = (sem_g0, sem_g1)
    sem_os = (sem_o0, sem_o1)
    nchunk = hpw // HC
    hist0 = pl.multiple_of(wid * hpw, hpw)

    pltpu.sync_copy(idx_hbm.at[pl.ds(hist0, hpw)], idx_all)

    @pl.loop(0, hpw)
    def _compact(ch):
        dst = ch * hist
        for o in (0, 16, 32, hist - 16):
            idx_cmp[pl.ds(dst + o, 16)] = idx_all[ch, pl.ds(o, 16)]

    ND = 5
    nd = HC * hist // ND

    def gather(c, b):
        for d in range(ND):
            off = pl.multiple_of(c * (HC * hist) + d * nd, nd)
            pltpu.async_copy(w_hbm.at[idx_cmp.at[pl.ds(off, nd)]],
                             rows_vs[b].at[pl.ds(d * nd, nd)], sem_gs[b])

    def wait_gather(b):
        for d in range(ND):
            pltpu.make_async_copy(w_hbm.at[pl.ds(0, nd)],
                                  rows_vs[b].at[pl.ds(d * nd, nd)],
                                  sem_gs[b]).wait()

    def wait_out(b):
        pltpu.make_async_copy(out_vs[b], out_hbm.at[pl.ds(0, HC)],
                              sem_os[b]).wait()

    gather(0, 0)

    @pl.loop(0, nchunk, step=2)
    def _pair(g):
        for b in range(2):
            c = g + b
            rows_v, out_v = rows_vs[b], out_vs[b]

            @pl.when(c + 1 < nchunk)
            def _pf_gather():
                gather(c + 1, 1 - b)

            wait_gather(b)

            @pl.when(c >= 2)
            def _drain_out():
                wait_out(b)

            for hh in range(HC):

                @pl.loop(0, hist // 5)
                def _rows(it):
                    r0 = hh * hist + it * 5
                    rng = range(5)
                    V = [[rows_v[r0 + j, pl.ds(16 * k, 16)]
                          for k in range(4)] for j in rng]
                    A = [[jnp.abs(x) for x in row] for row in V]
                    S1 = [jnp.maximum(a[0], a[1]) for a in A]
                    T1 = [jnp.minimum(a[0], a[1]) for a in A]
                    S2 = [jnp.maximum(a[2], a[3]) for a in A]
                    T2 = [jnp.minimum(a[2], a[3]) for a in A]
                    M1 = [jnp.maximum(x, y) for x, y in zip(S1, S2)]
                    M2 = [jnp.maximum(jnp.minimum(x, y),
                                      jnp.maximum(z, w))
                          for x, y, z, w in zip(S1, S2, T1, T2)]
                    KV = [plsc.sort_key_val(m1, m2, descending=True)
                          for m1, m2 in zip(M1, M2)]
                    M1s = [ks.at[zero16].get(mode="promise_in_bounds")
                           for ks, _ in KV]
                    K1s = [ks.at[one16].get(mode="promise_in_bounds")
                           for ks, _ in KV]
                    V0s = [vv.at[zero16].get(mode="promise_in_bounds")
                           for _, vv in KV]
                    M2s = [jnp.maximum(x, y) for x, y in zip(K1s, V0s)]
                    AM = [jnp.maximum(m2 + FRAC * (m1 - m2),
                                      np.float32(1e-8))
                          for m1, m2 in zip(M1s, M2s)]
                    INV = [np.float32(128.0) / am for am in AM]
                    NAM = [-am for am in AM]
                    for k in range(4):
                        X = [jnp.minimum(jnp.maximum(V[j][k], NAM[j]),
                                         AM[j]) for j in rng]
                        UF = [x * INV[j] + np.float32(128.0)
                              for j, x in zip(rng, X)]
                        U = [jnp.minimum(uf.astype(jnp.int32), 255)
                             for uf in UF]
                        MV = [plsc.load_gather(mvs_v, [u]) for u in U]
                        U2 = [u + u + jnp.where(uf > mv, 1, 0)
                              for u, uf, mv in zip(U, UF, MV)]
                        Q = [plsc.load_gather(flut_v, [u2]) for u2 in U2]
                        for j in rng:
                            out_v[hh, it * 5 + j, pl.ds(16 * k, 16)] = (
                                Q[j] * AM[j])

            h0 = pl.multiple_of(hist0 + c * HC, HC)
            pltpu.async_copy(out_v, out_hbm.at[pl.ds(h0, HC)], sem_os[b])

    wait_out(0)
    wait_out(1)


@functools.partial(jax.jit, static_argnums=(4,))
def _axs_embed(idx, weight, mvs, flut, hist):
    nhist = idx.shape[0]
    hpw = nhist // NW
    body = functools.partial(_sc_body, hpw=hpw, hist=hist)
    f = pl.kernel(
        body,
        out_type=jax.ShapeDtypeStruct((nhist, hist, D), jnp.float32),
        mesh=plsc.VectorSubcoreMesh(core_axis_name="c", subcore_axis_name="s",
                                    num_cores=2, num_subcores=16),
        scratch_types=[
            pltpu.VMEM((hpw, 128), jnp.int32),
            pltpu.VMEM((hpw * hist,), jnp.int32),
            pltpu.VMEM((HC * hist, D), jnp.float32),
            pltpu.VMEM((HC * hist, D), jnp.float32),
            pltpu.VMEM((HC, hist, D), jnp.float32),
            pltpu.VMEM((HC, hist, D), jnp.float32),
            pltpu.VMEM((256,), jnp.float32),
            pltpu.VMEM((512,), jnp.float32),
            pltpu.SemaphoreType.DMA,
            pltpu.SemaphoreType.DMA,
            pltpu.SemaphoreType.DMA,
            pltpu.SemaphoreType.DMA,
        ],
        compiler_params=pltpu.CompilerParams(needs_layout_passes=False,
                                             use_tc_tiling_on_sc=False),
    )
    return f(idx, weight, mvs, flut)


def kernel(input, weight):
    probs = (jnp.arange(32, dtype=jnp.float32) + 0.5) / 32
    lv = ndtri(probs)
    lv = (lv / jnp.max(jnp.abs(lv))).astype(jnp.float32)
    mids = (lv[:-1] + lv[1:]) * np.float32(0.5)
    midpad = jnp.concatenate([mids, jnp.full((1,), 2.0, jnp.float32)])
    edges = jnp.arange(256, dtype=jnp.float32) / np.float32(128.0) - 1
    lut = jnp.sum(mids[None, :] < edges[:, None], axis=1).astype(jnp.int32)
    mvs = (midpad[lut] + 1) * np.float32(128.0)
    flut = lv[jnp.minimum(lut[:, None] + jnp.arange(2)[None, :], 31)]
    idxp = jnp.pad(input, ((0, 0), (0, 128 - input.shape[1])))
    return _axs_embed(idxp, weight, mvs, flut.reshape(512), input.shape[1])


# speedup vs baseline: 1.0933x; 1.0933x over previous
import functools

import jax
import jax.numpy as jnp
import numpy as np
from jax import lax
from jax.experimental import pallas as pl
from jax.experimental.pallas import tpu as pltpu
from jax.experimental.pallas import tpu_sc as plsc
from jax.scipy.special import ndtri

D = 64
NW = 32
HC = 4
FRAC = np.float32(0.999 * 63 - 62)


def _sc_body(idx_hbm, w_hbm, mvs_hbm, flut_hbm, out_hbm,
             idx_all, idx_cmp, rows_v0, rows_v1, out_v0, out_v1,
             mvs_v, flut_v, sem_g0, sem_g1, sem_o0, sem_o1, hpw, hist):
    wid = lax.axis_index("s") * 2 + lax.axis_index("c")
    pltpu.sync_copy(mvs_hbm, mvs_v)
    pltpu.sync_copy(flut_hbm, flut_v)
    iota16 = lax.iota(jnp.int32, 16)
    zero16 = iota16 * 0
    one16 = zero16 + 1
    rows_vs = (rows_v0, rows_v1)
    out_vs = (out_v0, out_v1)
    sem_gs = (sem_g0, sem_g1)
    sem_os = (sem_o0, sem_o1)
    nchunk = hpw // HC
    hist0 = pl.multiple_of(wid * hpw, hpw)

    for rr in range(4):
        pltpu.sync_copy(idx_hbm.at[pl.ds(hist0 + rr * (hpw // 4), hpw // 4)],
                        idx_all)

        @pl.loop(0, hpw // 4)
        def _compact(ch):
            dst = (rr * (hpw // 4) + ch) * hist
            for o in (0, 16, 32, hist - 16):
                idx_cmp[pl.ds(dst + o, 16)] = idx_all[ch, pl.ds(o, 16)]

    ND = 5
    nd = HC * hist // ND

    def gather(c, b):
        for d in range(ND):
            off = pl.multiple_of(c * (HC * hist) + d * nd, nd)
            pltpu.async_copy(w_hbm.at[idx_cmp.at[pl.ds(off, nd)]],
                             rows_vs[b].at[pl.ds(d * nd, nd)], sem_gs[b])

    def wait_gather(b):
        for d in range(ND):
            pltpu.make_async_copy(w_hbm.at[pl.ds(0, nd)],
                                  rows_vs[b].at[pl.ds(d * nd, nd)],
                                  sem_gs[b]).wait()

    def wait_out(b):
        pltpu.make_async_copy(out_vs[b], out_hbm.at[pl.ds(0, HC)],
                              sem_os[b]).wait()

    gather(0, 0)

    @pl.loop(0, nchunk, step=2)
    def _pair(g):
        for b in range(2):
            c = g + b
            rows_v, out_v = rows_vs[b], out_vs[b]

            @pl.when(c + 1 < nchunk)
            def _pf_gather():
                gather(c + 1, 1 - b)

            wait_gather(b)

            @pl.when(c >= 2)
            def _drain_out():
                wait_out(b)

            for hh in range(HC):

                @pl.loop(0, hist // 5)
                def _rows(it):
                    r0 = hh * hist + it * 5
                    rng = range(5)
                    V = [[rows_v[r0 + j, pl.ds(16 * k, 16)]
                          for k in range(4)] for j in rng]
                    A = [[jnp.abs(x) for x in row] for row in V]
                    S1 = [jnp.maximum(a[0], a[1]) for a in A]
                    T1 = [jnp.minimum(a[0], a[1]) for a in A]
                    S2 = [jnp.maximum(a[2], a[3]) for a in A]
                    T2 = [jnp.minimum(a[2], a[3]) for a in A]
                    M1 = [jnp.maximum(x, y) for x, y in zip(S1, S2)]
                    M2 = [jnp.maximum(jnp.minimum(x, y),
                                      jnp.maximum(z, w))
                          for x, y, z, w in zip(S1, S2, T1, T2)]
                    KV = [plsc.sort_key_val(m1, m2, descending=True)
                          for m1, m2 in zip(M1, M2)]
                    M1s = [ks.at[zero16].get(mode="promise_in_bounds")
                           for ks, _ in KV]
                    K1s = [ks.at[one16].get(mode="promise_in_bounds")
                           for ks, _ in KV]
                    V0s = [vv.at[zero16].get(mode="promise_in_bounds")
                           for _, vv in KV]
                    M2s = [jnp.maximum(x, y) for x, y in zip(K1s, V0s)]
                    AM = [jnp.maximum(m2 + FRAC * (m1 - m2),
                                      np.float32(1e-8))
                          for m1, m2 in zip(M1s, M2s)]
                    INV = [np.float32(128.0) / am for am in AM]
                    NAM = [-am for am in AM]
                    for k in range(4):
                        X = [jnp.minimum(jnp.maximum(V[j][k], NAM[j]),
                                         AM[j]) for j in rng]
                        UF = [x * INV[j] + np.float32(128.0)
                              for j, x in zip(rng, X)]
                        U = [jnp.minimum(uf.astype(jnp.int32), 255)
                             for uf in UF]
                        MV = [plsc.load_gather(mvs_v, [u]) for u in U]
                        U2 = [u + u + jnp.where(uf > mv, 1, 0)
                              for u, uf, mv in zip(U, UF, MV)]
                        Q = [plsc.load_gather(flut_v, [u2]) for u2 in U2]
                        for j in rng:
                            out_v[hh, it * 5 + j, pl.ds(16 * k, 16)] = (
                                Q[j] * AM[j])

            h0 = pl.multiple_of(hist0 + c * HC, HC)
            pltpu.async_copy(out_v, out_hbm.at[pl.ds(h0, HC)], sem_os[b])

    wait_out(0)
    wait_out(1)


@functools.partial(jax.jit, static_argnums=(4,))
def _axs_embed(idx, weight, mvs, flut, hist):
    nhist = idx.shape[0]
    hpw = nhist // NW
    body = functools.partial(_sc_body, hpw=hpw, hist=hist)
    f = pl.kernel(
        body,
        out_type=jax.ShapeDtypeStruct((nhist, (hist + 7) // 8 * 8, 128),
                                      jnp.float32),
        mesh=plsc.VectorSubcoreMesh(core_axis_name="c", subcore_axis_name="s",
                                    num_cores=2, num_subcores=16),
        scratch_types=[
            pltpu.VMEM((hpw // 4, 128), jnp.int32),
            pltpu.VMEM((hpw * hist,), jnp.int32),
            pltpu.VMEM((HC * hist, D), jnp.float32),
            pltpu.VMEM((HC * hist, D), jnp.float32),
            pltpu.VMEM((HC, (hist + 7) // 8 * 8, 128), jnp.float32),
            pltpu.VMEM((HC, (hist + 7) // 8 * 8, 128), jnp.float32),
            pltpu.VMEM((256,), jnp.float32),
            pltpu.VMEM((512,), jnp.float32),
            pltpu.SemaphoreType.DMA,
            pltpu.SemaphoreType.DMA,
            pltpu.SemaphoreType.DMA,
            pltpu.SemaphoreType.DMA,
        ],
        compiler_params=pltpu.CompilerParams(needs_layout_passes=False,
                                             use_tc_tiling_on_sc=False),
    )
    return f(idx, weight, mvs, flut)


def kernel(input, weight):
    probs = (jnp.arange(32, dtype=jnp.float32) + 0.5) / 32
    lv = ndtri(probs)
    lv = (lv / jnp.max(jnp.abs(lv))).astype(jnp.float32)
    mids = (lv[:-1] + lv[1:]) * np.float32(0.5)
    midpad = jnp.concatenate([mids, jnp.full((1,), 2.0, jnp.float32)])
    edges = jnp.arange(256, dtype=jnp.float32) / np.float32(128.0) - 1
    lut = jnp.sum(mids[None, :] < edges[:, None], axis=1).astype(jnp.int32)
    mvs = (midpad[lut] + 1) * np.float32(128.0)
    flut = lv[jnp.minimum(lut[:, None] + jnp.arange(2)[None, :], 31)]
    idxp = jnp.pad(input, ((0, 0), (0, 128 - input.shape[1])))
    out = _axs_embed(idxp, weight, mvs, flut.reshape(512), input.shape[1])
    return out[:, :input.shape[1], :D]
